# baseline (speedup 1.0000x reference)
import functools

import jax
import jax.numpy as jnp
from jax import lax
from jax.experimental import pallas as pl
from jax.experimental.pallas import tpu as pltpu
from jax.experimental.pallas import tpu_sc as plsc

_N = 320000
_S = 5000
_SPAD = 5120
_NT = 16
_NA = _N // _NT
_C = 80
_NBC = 25
_BLK = _C * _NBC
_NBLK = _NA // _BLK


_R = 640
_Q = _N // 4


def _mlp_body(x0_ref, x1_ref, x2_ref, x3_ref, w1t_ref, b1_ref, w2_ref,
              b2_ref, o_ref):
    rows = []
    for xr in (x0_ref, x1_ref, x2_ref, x3_ref):
        h = jnp.dot(xr[...], w1t_ref[...], preferred_element_type=jnp.float32)
        h = h + b1_ref[...]
        h = h * jax.nn.sigmoid(h)
        rows.append(lax.dot_general(
            w2_ref[...], h, (((1,), (1,)), ((), ())),
            preferred_element_type=jnp.float32))
    o_ref[...] = jnp.concatenate(rows, axis=0) + b2_ref[0, 0]


def _mlp(x, W1, b1, W2, b2):
    n, hdim = x.shape
    hh = W1.shape[0]
    nb = _Q // _R

    def xspec(i):
        return pl.BlockSpec((_R, hdim), lambda g, i=i: (i * nb + g, 0))

    return pl.pallas_call(
        _mlp_body,
        grid=(nb,),
        in_specs=[
            xspec(0), xspec(1), xspec(2), xspec(3),
            pl.BlockSpec((hdim, hh), lambda g: (0, 0)),
            pl.BlockSpec((1, hh), lambda g: (0, 0)),
            pl.BlockSpec((1, hh), lambda g: (0, 0)),
            pl.BlockSpec((1, 1), lambda g: (0, 0)),
        ],
        out_specs=pl.BlockSpec((4, _R), lambda g: (0, g)),
        out_shape=jax.ShapeDtypeStruct((4, _Q), jnp.float32),
        compiler_params=pltpu.CompilerParams(
            dimension_semantics=("arbitrary",),
        ),
    )(x, x, x, x, W1.T, b1.reshape(1, hh), W2, b2.reshape(1, 1))


_mesh = plsc.VectorSubcoreMesh(core_axis_name="c", subcore_axis_name="s")
_NW = 2 * _NT
_NAW = _N // _NW
_NBLKW = _NAW // _BLK
_STRIPE = _SPAD // _NT


@functools.partial(
    pl.kernel,
    out_type=jax.ShapeDtypeStruct((2 * 4 * _SPAD,), jnp.float32),
    mesh=_mesh,
    scratch_types=[
        pltpu.VMEM((_SPAD,), jnp.float32),
        pltpu.VMEM((_SPAD,), jnp.float32),
        pltpu.VMEM((_SPAD,), jnp.float32),
        pltpu.VMEM((_SPAD,), jnp.float32),
        pltpu.VMEM_SHARED((_NT * _SPAD,), jnp.float32),
        pltpu.VMEM_SHARED((_NT * _SPAD,), jnp.float32),
        pltpu.VMEM_SHARED((_NT * _SPAD,), jnp.float32),
        pltpu.VMEM_SHARED((_NT * _SPAD,), jnp.float32),
        pltpu.VMEM((_SPAD,), jnp.float32),
        pltpu.VMEM((_SPAD,), jnp.float32),
        pltpu.VMEM((_SPAD,), jnp.float32),
        pltpu.VMEM((_SPAD,), jnp.float32),
        pltpu.VMEM((_STRIPE,), jnp.float32),
        pltpu.VMEM((_STRIPE,), jnp.float32),
        pltpu.VMEM((_STRIPE,), jnp.float32),
        pltpu.VMEM((_STRIPE,), jnp.float32),
        pltpu.VMEM((2 * _BLK,), jnp.int32),
        pltpu.VMEM((2 * _BLK,), jnp.int32),
        pltpu.VMEM((2 * _BLK,), jnp.float32),
        pltpu.VMEM((2 * _BLK,), jnp.float32),
        pltpu.VMEM((2 * _BLK,), jnp.float32),
        pltpu.VMEM((128,), jnp.float32),
        pltpu.SemaphoreType.DMA,
        pltpu.SemaphoreType.DMA,
        pltpu.SemaphoreType.DMA,
    ],
    compiler_params=pltpu.CompilerParams(needs_layout_passes=False),
)
def _sc_sums(z_hbm, b_hbm, px_hbm, py_hbm, pz_hbm, mass_hbm, part_hbm,
             apx, apy, apz, apm, stx, sty, stz, stm,
             mbx, mby, mbz, mbm, mcx, mcy, mcz, mcm,
             zblk, bblk, pxb, pyb, pzb, massr, msem, isem0, isem1):
    cid = lax.axis_index("c")
    sid = lax.axis_index("s")
    zero16 = jnp.zeros((16,), jnp.float32)
    isems = (isem0, isem1)

    pltpu.sync_copy(mass_hbm, massr)

    base = (cid * _NT + sid) * _NAW

    def fire(jb, par):
        a0 = base + jb * _BLK
        off = par * _BLK
        sem = isems[par]
        return [
            pltpu.async_copy(z_hbm.at[pl.ds(a0, _BLK)],
                             zblk.at[pl.ds(off, _BLK)], sem),
            pltpu.async_copy(b_hbm.at[pl.ds(a0, _BLK)],
                             bblk.at[pl.ds(off, _BLK)], sem),
            pltpu.async_copy(px_hbm.at[pl.ds(a0, _BLK)],
                             pxb.at[pl.ds(off, _BLK)], sem),
            pltpu.async_copy(py_hbm.at[pl.ds(a0, _BLK)],
                             pyb.at[pl.ds(off, _BLK)], sem),
            pltpu.async_copy(pz_hbm.at[pl.ds(a0, _BLK)],
                             pzb.at[pl.ds(off, _BLK)], sem),
        ]

    pend = [fire(0, 0), None]

    def zrow(i, carry):
        r0 = i * 16
        apx[pl.ds(r0, 16)] = zero16
        apy[pl.ds(r0, 16)] = zero16
        apz[pl.ds(r0, 16)] = zero16
        apm[pl.ds(r0, 16)] = zero16
        return carry

    lax.fori_loop(0, _SPAD // 16, zrow, 0)

    for jb in range(_NBLKW):
        par = jb % 2
        if jb + 1 < _NBLKW:
            pend[1 - par] = fire(jb + 1, 1 - par)
        for hd in pend[par]:
            hd.wait()
        off = par * _BLK

        def grp(i, carry, off=off):
            s = off + 16 * i
            zi = zblk[pl.ds(s, 16)]
            m = plsc.load_gather(massr, [zi])
            b = bblk[pl.ds(s, 16)]
            plsc.addupdate_scatter(apx, [b], m * pxb[pl.ds(s, 16)])
            plsc.addupdate_scatter(apy, [b], m * pyb[pl.ds(s, 16)])
            plsc.addupdate_scatter(apz, [b], m * pzb[pl.ds(s, 16)])
            plsc.addupdate_scatter(apm, [b], m)
            return carry

        lax.fori_loop(0, _BLK // 16, grp, 0)

    pltpu.sync_copy(apx, stx.at[pl.ds(sid * _SPAD, _SPAD)])
    pltpu.sync_copy(apy, sty.at[pl.ds(sid * _SPAD, _SPAD)])
    pltpu.sync_copy(apz, stz.at[pl.ds(sid * _SPAD, _SPAD)])
    pltpu.sync_copy(apm, stm.at[pl.ds(sid * _SPAD, _SPAD)])
    plsc.subcore_barrier()

    st0 = sid * _STRIPE
    handles = []
    for st, mb in ((stx, mbx), (sty, mby), (stz, mbz), (stm, mbm)):
        for t in range(_NT):
            handles.append(pltpu.async_copy(
                st.at[pl.ds(t * _SPAD + st0, _STRIPE)],
                mb.at[pl.ds(t * _STRIPE, _STRIPE)], msem))
    for hd in handles:
        hd.wait()

    def mrow(i, carry):
        r0 = i * 16
        sx = mbx[pl.ds(r0, 16)]
        sy = mby[pl.ds(r0, 16)]
        sz = mbz[pl.ds(r0, 16)]
        sm = mbm[pl.ds(r0, 16)]
        for t in range(1, _NT):
            sx = sx + mbx[pl.ds(t * _STRIPE + r0, 16)]
            sy = sy + mby[pl.ds(t * _STRIPE + r0, 16)]
            sz = sz + mbz[pl.ds(t * _STRIPE + r0, 16)]
            sm = sm + mbm[pl.ds(t * _STRIPE + r0, 16)]
        mcx[pl.ds(r0, 16)] = sx
        mcy[pl.ds(r0, 16)] = sy
        mcz[pl.ds(r0, 16)] = sz
        mcm[pl.ds(r0, 16)] = sm
        return carry

    lax.fori_loop(0, _STRIPE // 16, mrow, 0)

    pbase = cid * 4 * _SPAD
    pltpu.sync_copy(mcx, part_hbm.at[pl.ds(pbase + 0 * _SPAD + st0, _STRIPE)])
    pltpu.sync_copy(mcy, part_hbm.at[pl.ds(pbase + 1 * _SPAD + st0, _STRIPE)])
    pltpu.sync_copy(mcz, part_hbm.at[pl.ds(pbase + 2 * _SPAD + st0, _STRIPE)])
    pltpu.sync_copy(mcm, part_hbm.at[pl.ds(pbase + 3 * _SPAD + st0, _STRIPE)])


@functools.partial(
    pl.kernel,
    out_type=(
        jax.ShapeDtypeStruct((_N,), jnp.float32),
        jax.ShapeDtypeStruct((_N,), jnp.float32),
        jax.ShapeDtypeStruct((_N,), jnp.float32),
    ),
    mesh=_mesh,
    scratch_types=[
        pltpu.VMEM((_SPAD,), jnp.float32),
        pltpu.VMEM((_SPAD,), jnp.float32),
        pltpu.VMEM((_SPAD,), jnp.float32),
        pltpu.VMEM((_SPAD,), jnp.float32),
        pltpu.VMEM((_SPAD,), jnp.float32),
        pltpu.VMEM((_SPAD,), jnp.float32),
        pltpu.VMEM((_SPAD,), jnp.float32),
        pltpu.VMEM((2 * _BLK,), jnp.int32),
        pltpu.VMEM((2 * _BLK,), jnp.float32),
        pltpu.VMEM((2 * _BLK,), jnp.float32),
        pltpu.VMEM((2 * _BLK,), jnp.float32),
        pltpu.VMEM((2 * _BLK,), jnp.float32),
        pltpu.VMEM((2 * _BLK,), jnp.float32),
        pltpu.VMEM((2 * _BLK,), jnp.float32),
        pltpu.VMEM((2 * _BLK,), jnp.float32),
        pltpu.SemaphoreType.DMA,
        pltpu.SemaphoreType.DMA,
        pltpu.SemaphoreType.DMA,
        pltpu.SemaphoreType.DMA,
        pltpu.SemaphoreType.DMA,
    ],
    compiler_params=pltpu.CompilerParams(needs_layout_passes=False),
)
def _sc_out(part_hbm, b_hbm, px_hbm, py_hbm, pz_hbm, h_hbm,
            ox_hbm, oy_hbm, oz_hbm,
            cbx, cby, cbz, t0, t1, d0, d1,
            bblk, pxb, pyb, pzb, hblk, oxb, oyb, ozb,
            gsem, isem0, isem1, osem0, osem1):
    cid = lax.axis_index("c")
    sid = lax.axis_index("s")

    hs = [pltpu.async_copy(part_hbm.at[pl.ds(3 * _SPAD, _SPAD)], d0, gsem),
          pltpu.async_copy(part_hbm.at[pl.ds(7 * _SPAD, _SPAD)], d1, gsem)]
    for hd in hs:
        hd.wait()

    def drow(i, carry):
        r0 = i * 16
        sm = d0[pl.ds(r0, 16)] + d1[pl.ds(r0, 16)]
        d0[pl.ds(r0, 16)] = 1.0 / jnp.where(sm == 0.0, 1.0, sm)
        return carry

    lax.fori_loop(0, _SPAD // 16, drow, 0)

    for q, cb in ((0, cbx), (1, cby), (2, cbz)):
        hs = [pltpu.async_copy(
                  part_hbm.at[pl.ds(q * _SPAD, _SPAD)], t0, gsem),
              pltpu.async_copy(
                  part_hbm.at[pl.ds((4 + q) * _SPAD, _SPAD)], t1, gsem)]
        for hd in hs:
            hd.wait()

        def qrow(i, carry):
            r0 = i * 16
            cb[pl.ds(r0, 16)] = (
                (t0[pl.ds(r0, 16)] + t1[pl.ds(r0, 16)]) * d0[pl.ds(r0, 16)])
            return carry

        lax.fori_loop(0, _SPAD // 16, qrow, 0)

    base = (cid * _NT + sid) * _NAW
    isems = (isem0, isem1)
    osems = (osem0, osem1)

    def fire(jb, par):
        a0 = base + jb * _BLK
        off = par * _BLK
        sem = isems[par]
        return [
            pltpu.async_copy(b_hbm.at[pl.ds(a0, _BLK)],
                             bblk.at[pl.ds(off, _BLK)], sem),
            pltpu.async_copy(px_hbm.at[pl.ds(a0, _BLK)],
                             pxb.at[pl.ds(off, _BLK)], sem),
            pltpu.async_copy(py_hbm.at[pl.ds(a0, _BLK)],
                             pyb.at[pl.ds(off, _BLK)], sem),
            pltpu.async_copy(pz_hbm.at[pl.ds(a0, _BLK)],
                             pzb.at[pl.ds(off, _BLK)], sem),
            pltpu.async_copy(h_hbm.at[pl.ds(a0, _BLK)],
                             hblk.at[pl.ds(off, _BLK)], sem),
        ]

    pend = [fire(0, 0), None]
    opend = [None, None]

    for jb in range(_NBLKW):
        par = jb % 2
        if jb + 1 < _NBLKW:
            pend[1 - par] = fire(jb + 1, 1 - par)
        for hd in pend[par]:
            hd.wait()
        if opend[par] is not None:
            for hd in opend[par]:
                hd.wait()
        off = par * _BLK

        def grp(i, carry, off=off):
            s = off + 16 * i
            bi = bblk[pl.ds(s, 16)]
            hh = hblk[pl.ds(s, 16)]
            oxb[pl.ds(s, 16)] = hh * (
                pxb[pl.ds(s, 16)] - plsc.load_gather(cbx, [bi]))
            oyb[pl.ds(s, 16)] = hh * (
                pyb[pl.ds(s, 16)] - plsc.load_gather(cby, [bi]))
            ozb[pl.ds(s, 16)] = hh * (
                pzb[pl.ds(s, 16)] - plsc.load_gather(cbz, [bi]))
            return carry

        lax.fori_loop(0, _BLK // 16, grp, 0)
        a0 = base + jb * _BLK
        osem = osems[par]
        opend[par] = [
            pltpu.async_copy(oxb.at[pl.ds(off, _BLK)],
                             ox_hbm.at[pl.ds(a0, _BLK)], osem),
            pltpu.async_copy(oyb.at[pl.ds(off, _BLK)],
                             oy_hbm.at[pl.ds(a0, _BLK)], osem),
            pltpu.async_copy(ozb.at[pl.ds(off, _BLK)],
                             oz_hbm.at[pl.ds(a0, _BLK)], osem),
        ]

    for op in opend:
        if op is not None:
            for hd in op:
                hd.wait()


def kernel(x, v, z, pos, batch, W1, b1, W2, b2, atomic_mass):
    n = x.shape[0]
    zf = z.astype(jnp.int32)
    bf = batch.astype(jnp.int32)
    px, py, pz = pos[:, 0], pos[:, 1], pos[:, 2]
    massp = jnp.pad(atomic_mass, (0, 128 - atomic_mass.shape[0]))
    part = _sc_sums(zf, bf, px, py, pz, massp)
    h = _mlp(x, W1, b1, W2, b2).reshape(-1)
    ox, oy, oz = _sc_out(part, bf, px, py, pz, h)
    return jnp.stack([ox, oy, oz], axis=1)


# speedup vs baseline: 1.4999x; 1.4999x over previous
import functools

import jax
import jax.numpy as jnp
from jax import lax
from jax.experimental import pallas as pl
from jax.experimental.pallas import tpu as pltpu
from jax.experimental.pallas import tpu_sc as plsc

_N = 320000
_S = 5000
_SPAD = 5120
_NT = 16
_NA = _N // _NT
_C = 80
_NBC = 25
_BLK = _C * _NBC
_NBLK = _NA // _BLK


_R = 3200
_Q = _N // 4


def _mlp_body(x0_ref, x1_ref, x2_ref, x3_ref, w1t_ref, b1_ref, w2_ref,
              b2_ref, o_ref):
    rows = []
    for xr in (x0_ref, x1_ref, x2_ref, x3_ref):
        h = jnp.dot(xr[...], w1t_ref[...], preferred_element_type=jnp.float32)
        h = h + b1_ref[...]
        h = h * jax.nn.sigmoid(h)
        rows.append(lax.dot_general(
            w2_ref[...], h, (((1,), (1,)), ((), ())),
            preferred_element_type=jnp.float32))
    o_ref[...] = jnp.concatenate(rows, axis=0) + b2_ref[0, 0]


def _mlp(x, W1, b1, W2, b2):
    n, hdim = x.shape
    hh = W1.shape[0]
    nb = _Q // _R

    def xspec(i):
        return pl.BlockSpec((_R, hdim), lambda g, i=i: (i * nb + g, 0))

    return pl.pallas_call(
        _mlp_body,
        grid=(nb,),
        in_specs=[
            xspec(0), xspec(1), xspec(2), xspec(3),
            pl.BlockSpec((hdim, hh), lambda g: (0, 0)),
            pl.BlockSpec((1, hh), lambda g: (0, 0)),
            pl.BlockSpec((1, hh), lambda g: (0, 0)),
            pl.BlockSpec((1, 1), lambda g: (0, 0)),
        ],
        out_specs=pl.BlockSpec((4, _R), lambda g: (0, g)),
        out_shape=jax.ShapeDtypeStruct((4, _Q), jnp.float32),
        compiler_params=pltpu.CompilerParams(
            dimension_semantics=("arbitrary",),
        ),
    )(x, x, x, x, W1.T, b1.reshape(1, hh), W2, b2.reshape(1, 1))


_mesh = plsc.VectorSubcoreMesh(core_axis_name="c", subcore_axis_name="s")
_NW = 2 * _NT
_NAW = _N // _NW
_NBLKW = _NAW // _BLK
_STRIPE = _SPAD // _NT


@functools.partial(
    pl.kernel,
    out_type=jax.ShapeDtypeStruct((2 * 4 * _SPAD,), jnp.float32),
    mesh=_mesh,
    scratch_types=[
        pltpu.VMEM((_SPAD,), jnp.float32),
        pltpu.VMEM((_SPAD,), jnp.float32),
        pltpu.VMEM((_SPAD,), jnp.float32),
        pltpu.VMEM((_SPAD,), jnp.float32),
        pltpu.VMEM_SHARED((_NT * _SPAD,), jnp.float32),
        pltpu.VMEM_SHARED((_NT * _SPAD,), jnp.float32),
        pltpu.VMEM_SHARED((_NT * _SPAD,), jnp.float32),
        pltpu.VMEM_SHARED((_NT * _SPAD,), jnp.float32),
        pltpu.VMEM((_SPAD,), jnp.float32),
        pltpu.VMEM((_SPAD,), jnp.float32),
        pltpu.VMEM((_SPAD,), jnp.float32),
        pltpu.VMEM((_SPAD,), jnp.float32),
        pltpu.VMEM((_STRIPE,), jnp.float32),
        pltpu.VMEM((_STRIPE,), jnp.float32),
        pltpu.VMEM((_STRIPE,), jnp.float32),
        pltpu.VMEM((_STRIPE,), jnp.float32),
        pltpu.VMEM((2 * _BLK,), jnp.int32),
        pltpu.VMEM((2 * _BLK,), jnp.int32),
        pltpu.VMEM((2 * _BLK,), jnp.float32),
        pltpu.VMEM((2 * _BLK,), jnp.float32),
        pltpu.VMEM((2 * _BLK,), jnp.float32),
        pltpu.VMEM((128,), jnp.float32),
        pltpu.SemaphoreType.DMA,
        pltpu.SemaphoreType.DMA,
        pltpu.SemaphoreType.DMA,
    ],
    compiler_params=pltpu.CompilerParams(needs_layout_passes=False),
)
def _sc_sums(z_hbm, b_hbm, px_hbm, py_hbm, pz_hbm, mass_hbm, part_hbm,
             apx, apy, apz, apm, stx, sty, stz, stm,
             mbx, mby, mbz, mbm, mcx, mcy, mcz, mcm,
             zblk, bblk, pxb, pyb, pzb, massr, msem, isem0, isem1):
    cid = lax.axis_index("c")
    sid = lax.axis_index("s")
    zero16 = jnp.zeros((16,), jnp.float32)
    isems = (isem0, isem1)

    pltpu.sync_copy(mass_hbm, massr)

    base = (cid * _NT + sid) * _NAW

    def fire(jb, par):
        a0 = base + jb * _BLK
        off = par * _BLK
        sem = isems[par]
        return [
            pltpu.async_copy(z_hbm.at[pl.ds(a0, _BLK)],
                             zblk.at[pl.ds(off, _BLK)], sem),
            pltpu.async_copy(b_hbm.at[pl.ds(a0, _BLK)],
                             bblk.at[pl.ds(off, _BLK)], sem),
            pltpu.async_copy(px_hbm.at[pl.ds(a0, _BLK)],
                             pxb.at[pl.ds(off, _BLK)], sem),
            pltpu.async_copy(py_hbm.at[pl.ds(a0, _BLK)],
                             pyb.at[pl.ds(off, _BLK)], sem),
            pltpu.async_copy(pz_hbm.at[pl.ds(a0, _BLK)],
                             pzb.at[pl.ds(off, _BLK)], sem),
        ]

    pend = [fire(0, 0), None]

    def zrow(i, carry):
        r0 = i * 16
        apx[pl.ds(r0, 16)] = zero16
        apy[pl.ds(r0, 16)] = zero16
        apz[pl.ds(r0, 16)] = zero16
        apm[pl.ds(r0, 16)] = zero16
        return carry

    lax.fori_loop(0, _SPAD // 16, zrow, 0)

    for jb in range(_NBLKW):
        par = jb % 2
        if jb + 1 < _NBLKW:
            pend[1 - par] = fire(jb + 1, 1 - par)
        for hd in pend[par]:
            hd.wait()
        off = par * _BLK

        def grp(i, carry, off=off):
            s = off + 16 * i
            zi = zblk[pl.ds(s, 16)]
            m = plsc.load_gather(massr, [zi])
            b = bblk[pl.ds(s, 16)]
            plsc.addupdate_scatter(apx, [b], m * pxb[pl.ds(s, 16)])
            plsc.addupdate_scatter(apy, [b], m * pyb[pl.ds(s, 16)])
            plsc.addupdate_scatter(apz, [b], m * pzb[pl.ds(s, 16)])
            plsc.addupdate_scatter(apm, [b], m)
            return carry

        lax.fori_loop(0, _BLK // 16, grp, 0)

    pltpu.sync_copy(apx, stx.at[pl.ds(sid * _SPAD, _SPAD)])
    pltpu.sync_copy(apy, sty.at[pl.ds(sid * _SPAD, _SPAD)])
    pltpu.sync_copy(apz, stz.at[pl.ds(sid * _SPAD, _SPAD)])
    pltpu.sync_copy(apm, stm.at[pl.ds(sid * _SPAD, _SPAD)])
    plsc.subcore_barrier()

    st0 = sid * _STRIPE
    handles = []
    for st, mb in ((stx, mbx), (sty, mby), (stz, mbz), (stm, mbm)):
        for t in range(_NT):
            handles.append(pltpu.async_copy(
                st.at[pl.ds(t * _SPAD + st0, _STRIPE)],
                mb.at[pl.ds(t * _STRIPE, _STRIPE)], msem))
    for hd in handles:
        hd.wait()

    def mrow(i, carry):
        r0 = i * 16
        sx = mbx[pl.ds(r0, 16)]
        sy = mby[pl.ds(r0, 16)]
        sz = mbz[pl.ds(r0, 16)]
        sm = mbm[pl.ds(r0, 16)]
        for t in range(1, _NT):
            sx = sx + mbx[pl.ds(t * _STRIPE + r0, 16)]
            sy = sy + mby[pl.ds(t * _STRIPE + r0, 16)]
            sz = sz + mbz[pl.ds(t * _STRIPE + r0, 16)]
            sm = sm + mbm[pl.ds(t * _STRIPE + r0, 16)]
        mcx[pl.ds(r0, 16)] = sx
        mcy[pl.ds(r0, 16)] = sy
        mcz[pl.ds(r0, 16)] = sz
        mcm[pl.ds(r0, 16)] = sm
        return carry

    lax.fori_loop(0, _STRIPE // 16, mrow, 0)

    pbase = cid * 4 * _SPAD
    pltpu.sync_copy(mcx, part_hbm.at[pl.ds(pbase + 0 * _SPAD + st0, _STRIPE)])
    pltpu.sync_copy(mcy, part_hbm.at[pl.ds(pbase + 1 * _SPAD + st0, _STRIPE)])
    pltpu.sync_copy(mcz, part_hbm.at[pl.ds(pbase + 2 * _SPAD + st0, _STRIPE)])
    pltpu.sync_copy(mcm, part_hbm.at[pl.ds(pbase + 3 * _SPAD + st0, _STRIPE)])


@functools.partial(
    pl.kernel,
    out_type=(
        jax.ShapeDtypeStruct((_N,), jnp.float32),
        jax.ShapeDtypeStruct((_N,), jnp.float32),
        jax.ShapeDtypeStruct((_N,), jnp.float32),
    ),
    mesh=_mesh,
    scratch_types=[
        pltpu.VMEM((_SPAD,), jnp.float32),
        pltpu.VMEM((_SPAD,), jnp.float32),
        pltpu.VMEM((_SPAD,), jnp.float32),
        pltpu.VMEM((_SPAD,), jnp.float32),
        pltpu.VMEM((_SPAD,), jnp.float32),
        pltpu.VMEM((_SPAD,), jnp.float32),
        pltpu.VMEM((_SPAD,), jnp.float32),
        pltpu.VMEM((2 * _BLK,), jnp.int32),
        pltpu.VMEM((2 * _BLK,), jnp.float32),
        pltpu.VMEM((2 * _BLK,), jnp.float32),
        pltpu.VMEM((2 * _BLK,), jnp.float32),
        pltpu.VMEM((2 * _BLK,), jnp.float32),
        pltpu.VMEM((2 * _BLK,), jnp.float32),
        pltpu.VMEM((2 * _BLK,), jnp.float32),
        pltpu.VMEM((2 * _BLK,), jnp.float32),
        pltpu.SemaphoreType.DMA,
        pltpu.SemaphoreType.DMA,
        pltpu.SemaphoreType.DMA,
        pltpu.SemaphoreType.DMA,
        pltpu.SemaphoreType.DMA,
    ],
    compiler_params=pltpu.CompilerParams(needs_layout_passes=False),
)
def _sc_out(part_hbm, b_hbm, px_hbm, py_hbm, pz_hbm, h_hbm,
            ox_hbm, oy_hbm, oz_hbm,
            cbx, cby, cbz, t0, t1, d0, d1,
            bblk, pxb, pyb, pzb, hblk, oxb, oyb, ozb,
            gsem, isem0, isem1, osem0, osem1):
    cid = lax.axis_index("c")
    sid = lax.axis_index("s")

    hs = [pltpu.async_copy(part_hbm.at[pl.ds(3 * _SPAD, _SPAD)], d0, gsem),
          pltpu.async_copy(part_hbm.at[pl.ds(7 * _SPAD, _SPAD)], d1, gsem)]
    for hd in hs:
        hd.wait()

    def drow(i, carry):
        r0 = i * 16
        sm = d0[pl.ds(r0, 16)] + d1[pl.ds(r0, 16)]
        d0[pl.ds(r0, 16)] = 1.0 / jnp.where(sm == 0.0, 1.0, sm)
        return carry

    lax.fori_loop(0, _SPAD // 16, drow, 0)

    for q, cb in ((0, cbx), (1, cby), (2, cbz)):
        hs = [pltpu.async_copy(
                  part_hbm.at[pl.ds(q * _SPAD, _SPAD)], t0, gsem),
              pltpu.async_copy(
                  part_hbm.at[pl.ds((4 + q) * _SPAD, _SPAD)], t1, gsem)]
        for hd in hs:
            hd.wait()

        def qrow(i, carry):
            r0 = i * 16
            cb[pl.ds(r0, 16)] = (
                (t0[pl.ds(r0, 16)] + t1[pl.ds(r0, 16)]) * d0[pl.ds(r0, 16)])
            return carry

        lax.fori_loop(0, _SPAD // 16, qrow, 0)

    base = (cid * _NT + sid) * _NAW
    isems = (isem0, isem1)
    osems = (osem0, osem1)

    def fire(jb, par):
        a0 = base + jb * _BLK
        off = par * _BLK
        sem = isems[par]
        return [
            pltpu.async_copy(b_hbm.at[pl.ds(a0, _BLK)],
                             bblk.at[pl.ds(off, _BLK)], sem),
            pltpu.async_copy(px_hbm.at[pl.ds(a0, _BLK)],
                             pxb.at[pl.ds(off, _BLK)], sem),
            pltpu.async_copy(py_hbm.at[pl.ds(a0, _BLK)],
                             pyb.at[pl.ds(off, _BLK)], sem),
            pltpu.async_copy(pz_hbm.at[pl.ds(a0, _BLK)],
                             pzb.at[pl.ds(off, _BLK)], sem),
            pltpu.async_copy(h_hbm.at[pl.ds(a0, _BLK)],
                             hblk.at[pl.ds(off, _BLK)], sem),
        ]

    pend = [fire(0, 0), None]
    opend = [None, None]

    for jb in range(_NBLKW):
        par = jb % 2
        if jb + 1 < _NBLKW:
            pend[1 - par] = fire(jb + 1, 1 - par)
        for hd in pend[par]:
            hd.wait()
        if opend[par] is not None:
            for hd in opend[par]:
                hd.wait()
        off = par * _BLK

        def grp(i, carry, off=off):
            s = off + 16 * i
            bi = bblk[pl.ds(s, 16)]
            hh = hblk[pl.ds(s, 16)]
            oxb[pl.ds(s, 16)] = hh * (
                pxb[pl.ds(s, 16)] - plsc.load_gather(cbx, [bi]))
            oyb[pl.ds(s, 16)] = hh * (
                pyb[pl.ds(s, 16)] - plsc.load_gather(cby, [bi]))
            ozb[pl.ds(s, 16)] = hh * (
                pzb[pl.ds(s, 16)] - plsc.load_gather(cbz, [bi]))
            return carry

        lax.fori_loop(0, _BLK // 16, grp, 0)
        a0 = base + jb * _BLK
        osem = osems[par]
        opend[par] = [
            pltpu.async_copy(oxb.at[pl.ds(off, _BLK)],
                             ox_hbm.at[pl.ds(a0, _BLK)], osem),
            pltpu.async_copy(oyb.at[pl.ds(off, _BLK)],
                             oy_hbm.at[pl.ds(a0, _BLK)], osem),
            pltpu.async_copy(ozb.at[pl.ds(off, _BLK)],
                             oz_hbm.at[pl.ds(a0, _BLK)], osem),
        ]

    for op in opend:
        if op is not None:
            for hd in op:
                hd.wait()


def kernel(x, v, z, pos, batch, W1, b1, W2, b2, atomic_mass):
    n = x.shape[0]
    zf = z.astype(jnp.int32)
    bf = batch.astype(jnp.int32)
    px, py, pz = pos[:, 0], pos[:, 1], pos[:, 2]
    massp = jnp.pad(atomic_mass, (0, 128 - atomic_mass.shape[0]))
    part = _sc_sums(zf, bf, px, py, pz, massp)
    h = _mlp(x, W1, b1, W2, b2).reshape(-1)
    ox, oy, oz = _sc_out(part, bf, px, py, pz, h)
    return jnp.stack([ox, oy, oz], axis=1)


# speedup vs baseline: 1.6273x; 1.0850x over previous
import functools

import jax
import jax.numpy as jnp
from jax import lax
from jax.experimental import pallas as pl
from jax.experimental.pallas import tpu as pltpu
from jax.experimental.pallas import tpu_sc as plsc

_N = 320000
_S = 5000
_SPAD = 5120
_NT = 16
_NA = _N // _NT
_C = 80
_NBC = 25
_BLK = _C * _NBC
_NBLK = _NA // _BLK


_R = 3200
_Q = _N // 4


def _mlp_body(x0_ref, x1_ref, x2_ref, x3_ref, w1t_ref, b1_ref, w2_ref,
              b2_ref, o_ref):
    rows = []
    for xr in (x0_ref, x1_ref, x2_ref, x3_ref):
        h = jnp.dot(xr[...], w1t_ref[...], preferred_element_type=jnp.float32)
        h = h + b1_ref[...]
        h = h * jax.nn.sigmoid(h)
        rows.append(lax.dot_general(
            w2_ref[...], h, (((1,), (1,)), ((), ())),
            preferred_element_type=jnp.float32))
    o_ref[...] = jnp.concatenate(rows, axis=0) + b2_ref[0, 0]


def _mlp(x, W1, b1, W2, b2):
    n, hdim = x.shape
    hh = W1.shape[0]
    nb = _Q // _R

    def xspec(i):
        return pl.BlockSpec((_R, hdim), lambda g, i=i: (i * nb + g, 0))

    return pl.pallas_call(
        _mlp_body,
        grid=(nb,),
        in_specs=[
            xspec(0), xspec(1), xspec(2), xspec(3),
            pl.BlockSpec((hdim, hh), lambda g: (0, 0)),
            pl.BlockSpec((1, hh), lambda g: (0, 0)),
            pl.BlockSpec((1, hh), lambda g: (0, 0)),
            pl.BlockSpec((1, 1), lambda g: (0, 0)),
        ],
        out_specs=pl.BlockSpec((4, _R), lambda g: (0, g)),
        out_shape=jax.ShapeDtypeStruct((4, _Q), jnp.float32),
        compiler_params=pltpu.CompilerParams(
            dimension_semantics=("arbitrary",),
        ),
    )(x, x, x, x, W1.T, b1.reshape(1, hh), W2, b2.reshape(1, 1))


_mesh = plsc.VectorSubcoreMesh(core_axis_name="c", subcore_axis_name="s")
_NW = 2 * _NT
_NAW = _N // _NW
_NBLKW = _NAW // _BLK
_STRIPE = _SPAD // _NT


@functools.partial(
    pl.kernel,
    out_type=jax.ShapeDtypeStruct((2 * 4 * _SPAD,), jnp.float32),
    mesh=_mesh,
    scratch_types=[
        pltpu.VMEM((_SPAD,), jnp.float32),
        pltpu.VMEM((_SPAD,), jnp.float32),
        pltpu.VMEM((_SPAD,), jnp.float32),
        pltpu.VMEM((_SPAD,), jnp.float32),
        pltpu.VMEM_SHARED((_NT * _SPAD,), jnp.float32),
        pltpu.VMEM_SHARED((_NT * _SPAD,), jnp.float32),
        pltpu.VMEM_SHARED((_NT * _SPAD,), jnp.float32),
        pltpu.VMEM_SHARED((_NT * _SPAD,), jnp.float32),
        pltpu.VMEM((_SPAD,), jnp.float32),
        pltpu.VMEM((_SPAD,), jnp.float32),
        pltpu.VMEM((_SPAD,), jnp.float32),
        pltpu.VMEM((_SPAD,), jnp.float32),
        pltpu.VMEM((_STRIPE,), jnp.float32),
        pltpu.VMEM((_STRIPE,), jnp.float32),
        pltpu.VMEM((_STRIPE,), jnp.float32),
        pltpu.VMEM((_STRIPE,), jnp.float32),
        pltpu.VMEM((2 * _BLK,), jnp.int32),
        pltpu.VMEM((2 * _BLK,), jnp.int32),
        pltpu.VMEM((2 * _BLK,), jnp.float32),
        pltpu.VMEM((2 * _BLK,), jnp.float32),
        pltpu.VMEM((2 * _BLK,), jnp.float32),
        pltpu.VMEM((128,), jnp.float32),
        pltpu.SemaphoreType.DMA,
        pltpu.SemaphoreType.DMA,
        pltpu.SemaphoreType.DMA,
    ],
    compiler_params=pltpu.CompilerParams(needs_layout_passes=False),
)
def _sc_sums(z_hbm, b_hbm, px_hbm, py_hbm, pz_hbm, mass_hbm, part_hbm,
             apx, apy, apz, apm, stx, sty, stz, stm,
             mbx, mby, mbz, mbm, mcx, mcy, mcz, mcm,
             zblk, bblk, pxb, pyb, pzb, massr, msem, isem0, isem1):
    cid = lax.axis_index("c")
    sid = lax.axis_index("s")
    zero16 = jnp.zeros((16,), jnp.float32)
    isems = (isem0, isem1)

    pltpu.sync_copy(mass_hbm, massr)

    base = (cid * _NT + sid) * _NAW

    def fire(jb, par):
        a0 = base + jb * _BLK
        off = par * _BLK
        sem = isems[par]
        return [
            pltpu.async_copy(z_hbm.at[pl.ds(a0, _BLK)],
                             zblk.at[pl.ds(off, _BLK)], sem),
            pltpu.async_copy(b_hbm.at[pl.ds(a0, _BLK)],
                             bblk.at[pl.ds(off, _BLK)], sem),
            pltpu.async_copy(px_hbm.at[pl.ds(a0, _BLK)],
                             pxb.at[pl.ds(off, _BLK)], sem),
            pltpu.async_copy(py_hbm.at[pl.ds(a0, _BLK)],
                             pyb.at[pl.ds(off, _BLK)], sem),
            pltpu.async_copy(pz_hbm.at[pl.ds(a0, _BLK)],
                             pzb.at[pl.ds(off, _BLK)], sem),
        ]

    pend = [fire(0, 0), None]

    def zrow(i, carry):
        r0 = i * 16
        apx[pl.ds(r0, 16)] = zero16
        apy[pl.ds(r0, 16)] = zero16
        apz[pl.ds(r0, 16)] = zero16
        apm[pl.ds(r0, 16)] = zero16
        return carry

    lax.fori_loop(0, _SPAD // 16, zrow, 0)

    for jb in range(_NBLKW):
        par = jb % 2
        if jb + 1 < _NBLKW:
            pend[1 - par] = fire(jb + 1, 1 - par)
        for hd in pend[par]:
            hd.wait()
        off = par * _BLK

        def grp(i, carry, off=off):
            s = off + 16 * i
            zi = zblk[pl.ds(s, 16)]
            m = plsc.load_gather(massr, [zi])
            b = bblk[pl.ds(s, 16)]
            plsc.addupdate_scatter(apx, [b], m * pxb[pl.ds(s, 16)])
            plsc.addupdate_scatter(apy, [b], m * pyb[pl.ds(s, 16)])
            plsc.addupdate_scatter(apz, [b], m * pzb[pl.ds(s, 16)])
            plsc.addupdate_scatter(apm, [b], m)
            return carry

        lax.fori_loop(0, _BLK // 16, grp, 0)

    pltpu.sync_copy(apx, stx.at[pl.ds(sid * _SPAD, _SPAD)])
    pltpu.sync_copy(apy, sty.at[pl.ds(sid * _SPAD, _SPAD)])
    pltpu.sync_copy(apz, stz.at[pl.ds(sid * _SPAD, _SPAD)])
    pltpu.sync_copy(apm, stm.at[pl.ds(sid * _SPAD, _SPAD)])
    plsc.subcore_barrier()

    st0 = sid * _STRIPE
    handles = []
    for st, mb in ((stx, mbx), (sty, mby), (stz, mbz), (stm, mbm)):
        for t in range(_NT):
            handles.append(pltpu.async_copy(
                st.at[pl.ds(t * _SPAD + st0, _STRIPE)],
                mb.at[pl.ds(t * _STRIPE, _STRIPE)], msem))
    for hd in handles:
        hd.wait()

    def mrow(i, carry):
        r0 = i * 16
        sx = mbx[pl.ds(r0, 16)]
        sy = mby[pl.ds(r0, 16)]
        sz = mbz[pl.ds(r0, 16)]
        sm = mbm[pl.ds(r0, 16)]
        for t in range(1, _NT):
            sx = sx + mbx[pl.ds(t * _STRIPE + r0, 16)]
            sy = sy + mby[pl.ds(t * _STRIPE + r0, 16)]
            sz = sz + mbz[pl.ds(t * _STRIPE + r0, 16)]
            sm = sm + mbm[pl.ds(t * _STRIPE + r0, 16)]
        mcx[pl.ds(r0, 16)] = sx
        mcy[pl.ds(r0, 16)] = sy
        mcz[pl.ds(r0, 16)] = sz
        mcm[pl.ds(r0, 16)] = sm
        return carry

    lax.fori_loop(0, _STRIPE // 16, mrow, 0)

    pbase = cid * 4 * _SPAD
    pltpu.sync_copy(mcx, part_hbm.at[pl.ds(pbase + 0 * _SPAD + st0, _STRIPE)])
    pltpu.sync_copy(mcy, part_hbm.at[pl.ds(pbase + 1 * _SPAD + st0, _STRIPE)])
    pltpu.sync_copy(mcz, part_hbm.at[pl.ds(pbase + 2 * _SPAD + st0, _STRIPE)])
    pltpu.sync_copy(mcm, part_hbm.at[pl.ds(pbase + 3 * _SPAD + st0, _STRIPE)])


@functools.partial(
    pl.kernel,
    out_type=(
        jax.ShapeDtypeStruct((_N,), jnp.float32),
        jax.ShapeDtypeStruct((_N,), jnp.float32),
        jax.ShapeDtypeStruct((_N,), jnp.float32),
    ),
    mesh=_mesh,
    scratch_types=[
        pltpu.VMEM((_SPAD,), jnp.float32),
        pltpu.VMEM((_SPAD,), jnp.float32),
        pltpu.VMEM((_SPAD,), jnp.float32),
        pltpu.VMEM((3 * _STRIPE,), jnp.float32),
        pltpu.VMEM((3 * _STRIPE,), jnp.float32),
        pltpu.VMEM((_STRIPE,), jnp.float32),
        pltpu.VMEM((_STRIPE,), jnp.float32),
        pltpu.VMEM_SHARED((_SPAD,), jnp.float32),
        pltpu.VMEM_SHARED((_SPAD,), jnp.float32),
        pltpu.VMEM_SHARED((_SPAD,), jnp.float32),
        pltpu.VMEM((_STRIPE,), jnp.float32),
        pltpu.VMEM((_STRIPE,), jnp.float32),
        pltpu.VMEM((_STRIPE,), jnp.float32),
        pltpu.VMEM((2 * _BLK,), jnp.int32),
        pltpu.VMEM((2 * _BLK,), jnp.float32),
        pltpu.VMEM((2 * _BLK,), jnp.float32),
        pltpu.VMEM((2 * _BLK,), jnp.float32),
        pltpu.VMEM((2 * _BLK,), jnp.float32),
        pltpu.VMEM((2 * _BLK,), jnp.float32),
        pltpu.VMEM((2 * _BLK,), jnp.float32),
        pltpu.VMEM((2 * _BLK,), jnp.float32),
        pltpu.SemaphoreType.DMA,
        pltpu.SemaphoreType.DMA,
        pltpu.SemaphoreType.DMA,
        pltpu.SemaphoreType.DMA,
        pltpu.SemaphoreType.DMA,
    ],
    compiler_params=pltpu.CompilerParams(needs_layout_passes=False),
)
def _sc_out(part_hbm, b_hbm, px_hbm, py_hbm, pz_hbm, h_hbm,
            ox_hbm, oy_hbm, oz_hbm,
            cbx, cby, cbz, t0, t1, d0, d1, cshx, cshy, cshz, mcx, mcy, mcz,
            bblk, pxb, pyb, pzb, hblk, oxb, oyb, ozb,
            gsem, isem0, isem1, osem0, osem1):
    cid = lax.axis_index("c")
    sid = lax.axis_index("s")

    st0v = sid * _STRIPE
    hs = [
        pltpu.async_copy(
            part_hbm.at[pl.ds(3 * _SPAD + st0v, _STRIPE)],
            d0.at[pl.ds(0, _STRIPE)], gsem),
        pltpu.async_copy(
            part_hbm.at[pl.ds(7 * _SPAD + st0v, _STRIPE)],
            d1.at[pl.ds(0, _STRIPE)], gsem),
    ]
    for q in range(3):
        hs.append(pltpu.async_copy(
            part_hbm.at[pl.ds(q * _SPAD + st0v, _STRIPE)],
            t0.at[pl.ds(q * _STRIPE, _STRIPE)], gsem))
        hs.append(pltpu.async_copy(
            part_hbm.at[pl.ds((4 + q) * _SPAD + st0v, _STRIPE)],
            t1.at[pl.ds(q * _STRIPE, _STRIPE)], gsem))
    for hd in hs:
        hd.wait()

    def srow(i, carry):
        r0 = i * 16
        sm = d0[pl.ds(r0, 16)] + d1[pl.ds(r0, 16)]
        inv = 1.0 / jnp.where(sm == 0.0, 1.0, sm)
        for q, mc in ((0, mcx), (1, mcy), (2, mcz)):
            mc[pl.ds(r0, 16)] = (
                t0[pl.ds(q * _STRIPE + r0, 16)]
                + t1[pl.ds(q * _STRIPE + r0, 16)]) * inv
        return carry

    lax.fori_loop(0, _STRIPE // 16, srow, 0)

    pltpu.sync_copy(mcx, cshx.at[pl.ds(st0v, _STRIPE)])
    pltpu.sync_copy(mcy, cshy.at[pl.ds(st0v, _STRIPE)])
    pltpu.sync_copy(mcz, cshz.at[pl.ds(st0v, _STRIPE)])
    plsc.subcore_barrier()
    pltpu.sync_copy(cshx, cbx)
    pltpu.sync_copy(cshy, cby)
    pltpu.sync_copy(cshz, cbz)

    base = (cid * _NT + sid) * _NAW
    isems = (isem0, isem1)
    osems = (osem0, osem1)

    def fire(jb, par):
        a0 = base + jb * _BLK
        off = par * _BLK
        sem = isems[par]
        return [
            pltpu.async_copy(b_hbm.at[pl.ds(a0, _BLK)],
                             bblk.at[pl.ds(off, _BLK)], sem),
            pltpu.async_copy(px_hbm.at[pl.ds(a0, _BLK)],
                             pxb.at[pl.ds(off, _BLK)], sem),
            pltpu.async_copy(py_hbm.at[pl.ds(a0, _BLK)],
                             pyb.at[pl.ds(off, _BLK)], sem),
            pltpu.async_copy(pz_hbm.at[pl.ds(a0, _BLK)],
                             pzb.at[pl.ds(off, _BLK)], sem),
            pltpu.async_copy(h_hbm.at[pl.ds(a0, _BLK)],
                             hblk.at[pl.ds(off, _BLK)], sem),
        ]

    pend = [fire(0, 0), None]
    opend = [None, None]

    for jb in range(_NBLKW):
        par = jb % 2
        if jb + 1 < _NBLKW:
            pend[1 - par] = fire(jb + 1, 1 - par)
        for hd in pend[par]:
            hd.wait()
        if opend[par] is not None:
            for hd in opend[par]:
                hd.wait()
        off = par * _BLK

        def grp(i, carry, off=off):
            s = off + 16 * i
            bi = bblk[pl.ds(s, 16)]
            hh = hblk[pl.ds(s, 16)]
            oxb[pl.ds(s, 16)] = hh * (
                pxb[pl.ds(s, 16)] - plsc.load_gather(cbx, [bi]))
            oyb[pl.ds(s, 16)] = hh * (
                pyb[pl.ds(s, 16)] - plsc.load_gather(cby, [bi]))
            ozb[pl.ds(s, 16)] = hh * (
                pzb[pl.ds(s, 16)] - plsc.load_gather(cbz, [bi]))
            return carry

        lax.fori_loop(0, _BLK // 16, grp, 0)
        a0 = base + jb * _BLK
        osem = osems[par]
        opend[par] = [
            pltpu.async_copy(oxb.at[pl.ds(off, _BLK)],
                             ox_hbm.at[pl.ds(a0, _BLK)], osem),
            pltpu.async_copy(oyb.at[pl.ds(off, _BLK)],
                             oy_hbm.at[pl.ds(a0, _BLK)], osem),
            pltpu.async_copy(ozb.at[pl.ds(off, _BLK)],
                             oz_hbm.at[pl.ds(a0, _BLK)], osem),
        ]

    for op in opend:
        if op is not None:
            for hd in op:
                hd.wait()


def kernel(x, v, z, pos, batch, W1, b1, W2, b2, atomic_mass):
    n = x.shape[0]
    zf = z.astype(jnp.int32)
    bf = batch.astype(jnp.int32)
    px, py, pz = pos[:, 0], pos[:, 1], pos[:, 2]
    massp = jnp.pad(atomic_mass, (0, 128 - atomic_mass.shape[0]))
    part = _sc_sums(zf, bf, px, py, pz, massp)
    h = _mlp(x, W1, b1, W2, b2).reshape(-1)
    ox, oy, oz = _sc_out(part, bf, px, py, pz, h)
    return jnp.stack([ox, oy, oz], axis=1)
